# baseline (speedup 1.0000x reference)
import functools

import jax
import jax.numpy as jnp
from jax import lax
from jax.experimental import pallas as pl
from jax.experimental.pallas import tpu as pltpu
from jax.experimental.pallas import tpu_sc as plsc

AGE_LO_C = 20.0
INV_RANGE = 1.0 / 70.0
C1 = INV_RANGE * 9.99
MAGIC = 12582912.0
K_BITS = 0x4B400000
EPS_HALF = 0.49999997
CBS = -EPS_HALF - 20.0 * C1
_off = (-(K_BITS * 160 + (K_BITS << 4))) & 0xFFFFFFFF
IDX_OFF = _off - (1 << 32) if _off >= (1 << 31) else _off

N_TOTAL = 8388608
NC, NS, L = 2, 16, 16
NW = NC * NS
W_PER = N_TOTAL // NW

_MESH = plsc.VectorSubcoreMesh(
    core_axis_name="c", subcore_axis_name="s", num_cores=NC, num_subcores=NS)

_CP = pltpu.CompilerParams(needs_layout_passes=False)

U = 16
CHUNK = 16384
NCH = W_PER // CHUNK
LANE_CNT = float(W_PER // L)

OUT2_ROWS = 30
POLL_BOUND = 20000


def _worker_id():
    return lax.axis_index("s") * NC + lax.axis_index("c")


def _recip(v):
    i = plsc.bitcast(v, jnp.int32)
    r = plsc.bitcast(jnp.int32(0x7EF127EA) - i, jnp.float32)
    for _ in range(4):
        r = r * (2.0 - v * r)
    return r


def _tree(vals, op):
    t = list(vals)
    while len(t) > 1:
        t = [op(t[j], t[j + 1]) if j + 1 < len(t) else t[j]
             for j in range(0, len(t), 2)]
    return t[0]


@functools.partial(
    pl.kernel,
    out_type=(
        jax.ShapeDtypeStruct((NW, 2 * L), jnp.float32),
        jax.ShapeDtypeStruct((NW, OUT2_ROWS * L), jnp.float32),
        jax.ShapeDtypeStruct((L,), jnp.float32),
    ),
    mesh=_MESH,
    compiler_params=_CP,
    scratch_types=[
        pltpu.VMEM((2, CHUNK), jnp.float32),
        pltpu.VMEM((2, CHUNK), jnp.float32),
        pltpu.VMEM((NW, 2 * L), jnp.float32),
        pltpu.VMEM((200 * L,), jnp.float32),
        pltpu.VMEM((2 * L,), jnp.float32),
        pltpu.VMEM((OUT2_ROWS * L,), jnp.float32),
        pltpu.VMEM((NW, OUT2_ROWS * L), jnp.float32),
        pltpu.VMEM((L,), jnp.float32),
        pltpu.SemaphoreType.DMA,
        pltpu.SemaphoreType.DMA,
        pltpu.SemaphoreType.DMA,
        pltpu.SemaphoreType.DMA,
    ],
)
def _kmain(ages_hbm, loss_hbm, mm_hbm, part_hbm, res_hbm, ages_v, loss_v,
           mm_v, acc_v, mmst_v, st_v, part_v, res_v, sa0, sa1, sl0, sl1):
    wid = _worker_id()
    base = wid * W_PER
    sems_a = (sa0, sa1)
    sems_l = (sl0, sl1)

    for b in range(2):
        pltpu.async_copy(ages_hbm.at[pl.ds(base + b * CHUNK, CHUNK)],
                         ages_v.at[b], sems_a[b])

    def chunk_a(g, carry):
        mn, mx = carry
        for b in range(2):
            ch = 2 * g + b
            pltpu.make_async_copy(ages_hbm.at[pl.ds(base, CHUNK)],
                                  ages_v.at[b], sems_a[b]).wait()

            def vbody(i, c):
                mn_, mx_ = c
                a = ages_v[b, pl.ds(i * L, L)]
                return jnp.minimum(mn_, a), jnp.maximum(mx_, a)

            mn, mx = plsc.parallel_loop(
                0, CHUNK // L, unroll=U, carry=(mn, mx))(vbody)

            @pl.when(ch + 2 < NCH)
            def _():
                start = base + (ch + 2) * CHUNK
                pltpu.async_copy(ages_hbm.at[pl.ds(start, CHUNK)],
                                 ages_v.at[b], sems_a[b])
        return mn, mx

    mn0 = jnp.full((L,), 1e30, jnp.float32)
    mx0 = jnp.full((L,), -1e30, jnp.float32)
    mn_r, mx_r = lax.fori_loop(0, NCH // 2, chunk_a, (mn0, mx0))

    mmst_v[pl.ds(0, L)] = mn_r
    mmst_v[pl.ds(L, L)] = mx_r
    pltpu.sync_copy(mmst_v, mm_hbm.at[wid])

    def poll_a_cond(s):
        i, valid = s
        return jnp.logical_and(valid < NW, i < POLL_BOUND)

    def poll_a_body(s):
        i, _ = s
        pltpu.sync_copy(mm_hbm, mm_v)
        valid = jnp.int32(0)
        for w in range(NW):
            mnr = mm_v[w, pl.ds(0, L)]
            mxr = mm_v[w, pl.ds(L, L)]
            ok = jnp.all((mnr >= 19.0) & (mnr <= 91.0)
                         & (mxr >= 19.0) & (mxr <= 91.0))
            valid = valid + ok.astype(jnp.int32)
        return i + 1, valid

    lax.while_loop(poll_a_cond, poll_a_body, (jnp.int32(0), jnp.int32(0)))

    mn_raw = _tree([mm_v[w, pl.ds(0, L)] for w in range(NW)], jnp.minimum)
    mx_raw = _tree([mm_v[w, pl.ds(L, L)] for w in range(NW)], jnp.maximum)
    mn_x = jnp.clip((mn_raw - AGE_LO_C) * INV_RANGE, 0.0, 1.0)
    mx_x = jnp.clip((mx_raw - AGE_LO_C) * INV_RANGE, 0.0, 1.0)
    mn = jnp.min(mn_x)
    scale_v = _recip(jnp.broadcast_to(jnp.max(mx_x) - mn, (L,))) * 10.0
    c2_v = INV_RANGE * scale_v
    m2s_v = (20.0 * INV_RANGE) * scale_v + mn * scale_v + EPS_HALF

    zeros = jnp.zeros((L,), jnp.float32)
    for k in range(200):
        acc_v[pl.ds(k * L, L)] = zeros

    for b in range(2):
        pltpu.async_copy(ages_hbm.at[pl.ds(base + b * CHUNK, CHUNK)],
                         ages_v.at[b], sems_a[b])
        pltpu.async_copy(loss_hbm.at[pl.ds(base + b * CHUNK, CHUNK)],
                         loss_v.at[b], sems_l[b])

    lane_off = lax.iota(jnp.int32, L) + jnp.int32(IDX_OFF)
    ones = jnp.ones((L,), jnp.float32)

    def chunk_b(g, carry):
        for b in range(2):
            ch = 2 * g + b
            pltpu.make_async_copy(ages_hbm.at[pl.ds(base, CHUNK)],
                                  ages_v.at[b], sems_a[b]).wait()
            pltpu.make_async_copy(loss_hbm.at[pl.ds(base, CHUNK)],
                                  loss_v.at[b], sems_l[b]).wait()

            @plsc.parallel_loop(0, CHUNK // L, unroll=U)
            def _(i):
                off = i * L
                a = ages_v[b, pl.ds(off, L)]
                bmag = (a * C1 + CBS) + MAGIC
                hsm = a * c2_v - m2s_v
                hmag = jnp.minimum(hsm, 9.0) + MAGIC
                fl = (plsc.bitcast(bmag, jnp.int32) * 160
                      + (plsc.bitcast(hmag, jnp.int32) << 4) + lane_off)
                plsc.addupdate_scatter(acc_v, [fl], ones)
                lvv = loss_v[b, pl.ds(off, L)]
                plsc.addupdate_scatter(acc_v, [fl + 100 * L], lvv)

            @pl.when(ch + 2 < NCH)
            def _():
                start = base + (ch + 2) * CHUNK
                pltpu.async_copy(ages_hbm.at[pl.ds(start, CHUNK)],
                                 ages_v.at[b], sems_a[b])
                pltpu.async_copy(loss_hbm.at[pl.ds(start, CHUNK)],
                                 loss_v.at[b], sems_l[b])
        return carry

    lax.fori_loop(0, NCH // 2, chunk_b, 0)

    for b10 in range(10):
        c2 = _tree([acc_v[pl.ds((b10 * 10 + hb) * L, L)] for hb in range(10)],
                   jnp.add)
        sl = _tree([acc_v[pl.ds((100 + b10 * 10 + hb) * L, L)]
                    for hb in range(10)], jnp.add)
        st_v[pl.ds(b10 * L, L)] = c2
        st_v[pl.ds((10 + b10) * L, L)] = sl
    for hb in range(10):
        hh = _tree([acc_v[pl.ds((b10 * 10 + hb) * L, L)] for b10 in range(10)],
                   jnp.add)
        st_v[pl.ds((20 + hb) * L, L)] = hh
    pltpu.sync_copy(st_v, part_hbm.at[wid])

    @pl.when(wid == 0)
    def _():
        def poll_c_cond(s):
            i, valid = s
            return jnp.logical_and(valid < NW, i < POLL_BOUND)

        def poll_c_body(s):
            i, _ = s
            pltpu.sync_copy(part_hbm, part_v)
            valid = jnp.int32(0)
            for w in range(NW):
                tot = _tree([part_v[w, pl.ds(k * L, L)] for k in range(10)],
                            jnp.add)
                ok = jnp.all(tot == LANE_CNT)
                valid = valid + ok.astype(jnp.int32)
            return i + 1, valid

        lax.while_loop(poll_c_cond, poll_c_body,
                       (jnp.int32(0), jnp.int32(0)))

        lane = lax.iota(jnp.int32, L)
        hist_t = jnp.zeros((L,), jnp.float32)
        c2_t = jnp.zeros((L,), jnp.float32)
        sl_t = jnp.zeros((L,), jnp.float32)
        for k in range(10):
            c2 = _tree([part_v[w, pl.ds(k * L, L)] for w in range(NW)],
                       jnp.add)
            sl = _tree([part_v[w, pl.ds((10 + k) * L, L)] for w in range(NW)],
                       jnp.add)
            hh = _tree([part_v[w, pl.ds((20 + k) * L, L)] for w in range(NW)],
                       jnp.add)
            onehot = lane == k
            hist_t = hist_t + jnp.where(onehot, jnp.sum(hh), 0.0)
            c2_t = c2_t + jnp.where(onehot, jnp.sum(c2), 0.0)
            sl_t = sl_t + jnp.where(onehot, jnp.sum(sl), 0.0)
        rec = _recip(hist_t + 1e-6)
        num = jnp.sum(sl_t * rec)
        den = jnp.sum(c2_t * rec)
        res_v[...] = num * _recip(jnp.broadcast_to(den, (L,)))
        pltpu.sync_copy(res_v, res_hbm)


def kernel(ages, base_loss):
    if base_loss.size == 1:
        return base_loss
    a = ages.reshape(-1)
    _, _, out = _kmain(a, base_loss)
    return out[0]


# speedup vs baseline: 282.5215x; 282.5215x over previous
import functools

import jax
import jax.numpy as jnp
from jax import lax
from jax.experimental import pallas as pl
from jax.experimental.pallas import tpu as pltpu
from jax.experimental.pallas import tpu_sc as plsc

AGE_LO_C = 20.0
INV_RANGE = 1.0 / 70.0
C1 = INV_RANGE * 9.99
MAGIC = 12582912.0
K_BITS = 0x4B400000
EPS_HALF = 0.49999997
CBS = -EPS_HALF - 20.0 * C1
_off = (-(K_BITS * 160 + (K_BITS << 4))) & 0xFFFFFFFF
IDX_OFF = _off - (1 << 32) if _off >= (1 << 31) else _off
N_TOTAL = 8388608
NC, NS, L = 2, 16, 16
NW = NC * NS
W_PER = N_TOTAL // NW

_MESH = plsc.VectorSubcoreMesh(
    core_axis_name="c", subcore_axis_name="s", num_cores=NC, num_subcores=NS)

_CP = pltpu.CompilerParams(needs_layout_passes=False)

U = 16
CHUNK1 = 32768
NCH1 = W_PER // CHUNK1
CHUNK2 = 16384
NCH2 = W_PER // CHUNK2

OUT2_ROWS = 30


def _worker_id():
    return lax.axis_index("s") * NC + lax.axis_index("c")


def _recip(v):
    i = plsc.bitcast(v, jnp.int32)
    r = plsc.bitcast(jnp.int32(0x7EF127EA) - i, jnp.float32)
    for _ in range(4):
        r = r * (2.0 - v * r)
    return r


def _tree(vals, op):
    t = list(vals)
    while len(t) > 1:
        t = [op(t[j], t[j + 1]) if j + 1 < len(t) else t[j]
             for j in range(0, len(t), 2)]
    return t[0]


@functools.partial(
    pl.kernel,
    out_type=jax.ShapeDtypeStruct((NW, 2 * L), jnp.float32),
    mesh=_MESH,
    compiler_params=_CP,
    scratch_types=[
        pltpu.VMEM((2, CHUNK1), jnp.float32),
        pltpu.VMEM((2 * L,), jnp.float32),
        pltpu.SemaphoreType.DMA,
        pltpu.SemaphoreType.DMA,
    ],
)
def _k1(ages_hbm, out_hbm, ages_v, mm_v, sa0, sa1):
    wid = _worker_id()
    base = wid * W_PER
    sems_a = (sa0, sa1)

    for b in range(2):
        pltpu.async_copy(ages_hbm.at[pl.ds(base + b * CHUNK1, CHUNK1)],
                         ages_v.at[b], sems_a[b])

    nb1 = CHUNK1 // (L * U)

    def chunk_body(g, carry):
        mn, mx = carry
        for b in range(2):
            ch = 2 * g + b
            pltpu.make_async_copy(ages_hbm.at[pl.ds(base, CHUNK1)],
                                  ages_v.at[b], sems_a[b]).wait()

            def vbody(i, c):
                mn_, mx_ = c
                a = ages_v[b, pl.ds(i * L, L)]
                return jnp.minimum(mn_, a), jnp.maximum(mx_, a)

            mn, mx = plsc.parallel_loop(
                0, CHUNK1 // L, unroll=U, carry=(mn, mx))(vbody)

            @pl.when(ch + 2 < NCH1)
            def _():
                start = base + (ch + 2) * CHUNK1
                pltpu.async_copy(ages_hbm.at[pl.ds(start, CHUNK1)],
                                 ages_v.at[b], sems_a[b])
        return mn, mx

    mn0 = jnp.full((L,), 1e30, jnp.float32)
    mx0 = jnp.full((L,), -1e30, jnp.float32)
    mn, mx = lax.fori_loop(0, NCH1 // 2, chunk_body, (mn0, mx0))

    mm_v[pl.ds(0, L)] = mn
    mm_v[pl.ds(L, L)] = mx
    pltpu.sync_copy(mm_v, out_hbm.at[wid])


@functools.partial(
    pl.kernel,
    out_type=jax.ShapeDtypeStruct((NW, OUT2_ROWS * L), jnp.float32),
    mesh=_MESH,
    compiler_params=_CP,
    scratch_types=[
        pltpu.VMEM((2, CHUNK2), jnp.float32),
        pltpu.VMEM((2, CHUNK2), jnp.float32),
        pltpu.VMEM((NW, 2 * L), jnp.float32),
        pltpu.VMEM((200 * L,), jnp.float32),
        pltpu.VMEM((OUT2_ROWS * L,), jnp.float32),
        pltpu.SemaphoreType.DMA,
        pltpu.SemaphoreType.DMA,
        pltpu.SemaphoreType.DMA,
        pltpu.SemaphoreType.DMA,
    ],
)
def _k2(ages_hbm, loss_hbm, mm_hbm, out_hbm, ages_v, loss_v, mm_v, acc_v,
        st_v, sa0, sa1, sl0, sl1):
    wid = _worker_id()
    base = wid * W_PER
    sems_a = (sa0, sa1)
    sems_l = (sl0, sl1)

    pltpu.sync_copy(mm_hbm, mm_v)
    mn_raw = _tree([mm_v[w, pl.ds(0, L)] for w in range(NW)], jnp.minimum)
    mx_raw = _tree([mm_v[w, pl.ds(L, L)] for w in range(NW)], jnp.maximum)
    mn_x = jnp.clip((mn_raw - AGE_LO_C) * INV_RANGE, 0.0, 1.0)
    mx_x = jnp.clip((mx_raw - AGE_LO_C) * INV_RANGE, 0.0, 1.0)
    mn = jnp.min(mn_x)
    scale_v = _recip(jnp.broadcast_to(jnp.max(mx_x) - mn, (L,))) * 10.0
    c2_v = INV_RANGE * scale_v
    m2s_v = (20.0 * INV_RANGE) * scale_v + mn * scale_v + EPS_HALF

    zeros = jnp.zeros((L,), jnp.float32)
    for k in range(200):
        acc_v[pl.ds(k * L, L)] = zeros

    for b in range(2):
        pltpu.async_copy(ages_hbm.at[pl.ds(base + b * CHUNK2, CHUNK2)],
                         ages_v.at[b], sems_a[b])
        pltpu.async_copy(loss_hbm.at[pl.ds(base + b * CHUNK2, CHUNK2)],
                         loss_v.at[b], sems_l[b])

    lane_off = lax.iota(jnp.int32, L) + jnp.int32(IDX_OFF)
    ones = jnp.ones((L,), jnp.float32)

    def chunk_body(g, carry):
        for b in range(2):
            ch = 2 * g + b
            pltpu.make_async_copy(ages_hbm.at[pl.ds(base, CHUNK2)],
                                  ages_v.at[b], sems_a[b]).wait()
            pltpu.make_async_copy(loss_hbm.at[pl.ds(base, CHUNK2)],
                                  loss_v.at[b], sems_l[b]).wait()

            @plsc.parallel_loop(0, CHUNK2 // L, unroll=U)
            def _(i):
                off = i * L
                a = ages_v[b, pl.ds(off, L)]
                bmag = (a * C1 + CBS) + MAGIC
                hsm = a * c2_v - m2s_v
                hmag = jnp.minimum(hsm, 9.0) + MAGIC
                fl = (plsc.bitcast(bmag, jnp.int32) * 160
                      + (plsc.bitcast(hmag, jnp.int32) << 4) + lane_off)
                plsc.addupdate_scatter(acc_v, [fl], ones)
                lvv = loss_v[b, pl.ds(off, L)]
                plsc.addupdate_scatter(acc_v, [fl + 100 * L], lvv)

            @pl.when(ch + 2 < NCH2)
            def _():
                start = base + (ch + 2) * CHUNK2
                pltpu.async_copy(ages_hbm.at[pl.ds(start, CHUNK2)],
                                 ages_v.at[b], sems_a[b])
                pltpu.async_copy(loss_hbm.at[pl.ds(start, CHUNK2)],
                                 loss_v.at[b], sems_l[b])
        return carry

    lax.fori_loop(0, NCH2 // 2, chunk_body, 0)

    for b10 in range(10):
        c2 = _tree([acc_v[pl.ds((b10 * 10 + hb) * L, L)] for hb in range(10)],
                   jnp.add)
        sl = _tree([acc_v[pl.ds((100 + b10 * 10 + hb) * L, L)]
                    for hb in range(10)], jnp.add)
        st_v[pl.ds(b10 * L, L)] = c2
        st_v[pl.ds((10 + b10) * L, L)] = sl
    for hb in range(10):
        hh = _tree([acc_v[pl.ds((b10 * 10 + hb) * L, L)] for b10 in range(10)],
                   jnp.add)
        st_v[pl.ds((20 + hb) * L, L)] = hh
    pltpu.sync_copy(st_v, out_hbm.at[wid])


@functools.partial(
    pl.kernel,
    out_type=jax.ShapeDtypeStruct((L,), jnp.float32),
    mesh=_MESH,
    compiler_params=_CP,
    scratch_types=[
        pltpu.VMEM((NW, OUT2_ROWS * L), jnp.float32),
        pltpu.VMEM((L,), jnp.float32),
    ],
)
def _k3(part_hbm, out_hbm, part_v, res_v):
    wid = _worker_id()

    @pl.when(wid == 0)
    def _():
        pltpu.sync_copy(part_hbm, part_v)
        lane = lax.iota(jnp.int32, L)
        zeros = jnp.zeros((L,), jnp.float32)
        hist_t = zeros
        c2_t = zeros
        sl_t = zeros
        for k in range(10):
            c2 = _tree([part_v[w, pl.ds(k * L, L)] for w in range(NW)], jnp.add)
            sl = _tree([part_v[w, pl.ds((10 + k) * L, L)] for w in range(NW)],
                       jnp.add)
            hh = _tree([part_v[w, pl.ds((20 + k) * L, L)] for w in range(NW)],
                       jnp.add)
            onehot = lane == k
            hist_t = hist_t + jnp.where(onehot, jnp.sum(hh), 0.0)
            c2_t = c2_t + jnp.where(onehot, jnp.sum(c2), 0.0)
            sl_t = sl_t + jnp.where(onehot, jnp.sum(sl), 0.0)
        rec = _recip(hist_t + 1e-6)
        num = jnp.sum(sl_t * rec)
        den = jnp.sum(c2_t * rec)
        res_v[...] = num * _recip(jnp.broadcast_to(den, (L,)))
        pltpu.sync_copy(res_v, out_hbm)


def kernel(ages, base_loss):
    if base_loss.size == 1:
        return base_loss
    a = ages.reshape(-1)
    mm = _k1(a)
    part = _k2(a, base_loss, mm)
    out = _k3(part)
    return out[0]
